# baseline (speedup 1.0000x reference)
import functools

import jax
import jax.numpy as jnp
from jax import lax
from jax.experimental import pallas as pl
from jax.experimental.pallas import tpu as pltpu
from jax.experimental.pallas import tpu_sc as plsc

NUM_CATEGORIES = 100000
D = 128
B = 16384
LANES = 16
VPR = D // LANES

_info = plsc.get_sparse_core_info()
NC, NS = _info.num_cores, _info.num_subcores
NW = NC * NS
B_PER_W = B // NW
CHUNK = 128
NCHUNKS = B_PER_W // CHUNK


def _rsqrt_vec(ss):
    i = lax.bitcast_convert_type(ss, jnp.int32)
    i = jnp.full((LANES,), 0x5F3759DF, jnp.int32) - lax.shift_right_logical(i, 1)
    y = lax.bitcast_convert_type(i, jnp.float32)
    half = ss * 0.5
    for _ in range(1):
        y = y * (1.5 - half * y * y)
    return y


_GATHER_DNUMS = lax.GatherDimensionNumbers(
    offset_dims=(), collapsed_slice_dims=(0,), start_index_map=(0,)
)


def _shuffle(v, idx):
    return lax.gather(
        v, idx[:, None], _GATHER_DNUMS, (1,),
        mode=lax.GatherScatterMode.PROMISE_IN_BOUNDS,
    )


def _lane_sum(v):
    iota = lax.iota(jnp.int32, LANES)
    for k in (8, 4, 2, 1):
        v = v + _shuffle(v, iota ^ k)
    return v


def _normalize_chunk(src, dst):
    def row(i, carry):
        acc = jnp.zeros((LANES,), jnp.float32)
        vs = []
        for j in range(VPR):
            v = src[i, pl.ds(j * LANES, LANES)]
            vs.append(v)
            acc = acc + v * v
        ss = _lane_sum(acc)
        inv = _rsqrt_vec(ss)
        for j in range(VPR):
            dst[i, pl.ds(j * LANES, LANES)] = vs[j] * inv
        return carry

    lax.fori_loop(0, CHUNK, row, 0, unroll=4)


def _sc_body(x_hbm, table_hbm, out_hbm, idx_v, *bufs_and_sems):
    bufs = bufs_and_sems[:NCHUNKS]
    obufs = bufs_and_sems[NCHUNKS:NCHUNKS + 2]
    gsems = bufs_and_sems[NCHUNKS + 2:2 * NCHUNKS + 2]
    wsems = bufs_and_sems[2 * NCHUNKS + 2:]
    wid = lax.axis_index("s") * NC + lax.axis_index("c")
    base = wid * B_PER_W
    pltpu.sync_copy(x_hbm.at[pl.ds(base, B_PER_W)], idx_v)
    gh = [
        pltpu.async_copy(
            table_hbm.at[idx_v.at[pl.ds(c * CHUNK, CHUNK)]], bufs[c], gsems[c])
        for c in range(NCHUNKS)
    ]
    wh = [None, None]
    for c in range(NCHUNKS):
        gh[c].wait()
        p = c % 2
        if wh[p] is not None:
            wh[p].wait()
        _normalize_chunk(bufs[c], obufs[p])
        wh[p] = pltpu.async_copy(
            obufs[p], out_hbm.at[pl.ds(base + c * CHUNK, CHUNK)], wsems[p])
    for h in wh:
        if h is not None:
            h.wait()


@jax.jit
def kernel(x, table):
    mesh = plsc.VectorSubcoreMesh(core_axis_name="c", subcore_axis_name="s")
    k = functools.partial(
        pl.kernel,
        mesh=mesh,
        out_type=jax.ShapeDtypeStruct((B, D), jnp.float32),
        scratch_types=(
            [pltpu.VMEM((B_PER_W,), jnp.int32)]
            + [pltpu.VMEM((CHUNK, D), jnp.float32)] * (NCHUNKS + 2)
            + [pltpu.SemaphoreType.DMA] * (NCHUNKS + 2)
        ),
    )(_sc_body)
    return k(x.astype(jnp.int32), table)


# speedup vs baseline: 1.4808x; 1.4808x over previous
import functools

import jax
import jax.numpy as jnp
from jax import lax
from jax.experimental import pallas as pl
from jax.experimental.pallas import tpu as pltpu
from jax.experimental.pallas import tpu_sc as plsc

NUM_CATEGORIES = 100000
D = 128
B = 16384
LANES = 16
VPR = D // LANES

_info = plsc.get_sparse_core_info()
NC, NS = _info.num_cores, _info.num_subcores
NW = NC * NS
B_PER_W = B // NW
CHUNK = 128
NCHUNKS = B_PER_W // CHUNK


def _rsqrt_vec(ss):
    i = lax.bitcast_convert_type(ss, jnp.int32)
    i = jnp.full((LANES,), 0x5F3759DF, jnp.int32) - lax.shift_right_logical(i, 1)
    y = lax.bitcast_convert_type(i, jnp.float32)
    half = ss * 0.5
    for _ in range(1):
        y = y * (1.5 - half * y * y)
    return y


_GATHER_DNUMS = lax.GatherDimensionNumbers(
    offset_dims=(), collapsed_slice_dims=(0,), start_index_map=(0,)
)


def _shuffle(v, idx):
    return lax.gather(
        v, idx[:, None], _GATHER_DNUMS, (1,),
        mode=lax.GatherScatterMode.PROMISE_IN_BOUNDS,
    )


def _lane_sum(v):
    iota = lax.iota(jnp.int32, LANES)
    for k in (8, 4, 2, 1):
        v = v + _shuffle(v, iota ^ k)
    return v


def _normalize_chunk(buf):
    def row(i, carry):
        acc = jnp.zeros((LANES,), jnp.float32)
        vs = []
        for j in range(VPR):
            v = buf[i, pl.ds(j * LANES, LANES)]
            vs.append(v)
            acc = acc + v * v
        ss = _lane_sum(acc)
        inv = _rsqrt_vec(ss)
        for j in range(VPR):
            buf[i, pl.ds(j * LANES, LANES)] = vs[j] * inv
        return carry

    lax.fori_loop(0, CHUNK, row, 0, unroll=4)


def _sc_body(x_hbm, table_hbm, out_hbm, idx_v, *bufs_and_sems):
    bufs = bufs_and_sems[:NCHUNKS]
    gsems = bufs_and_sems[NCHUNKS:2 * NCHUNKS]
    wsems = bufs_and_sems[2 * NCHUNKS:]
    wid = lax.axis_index("s") * NC + lax.axis_index("c")
    base = wid * B_PER_W
    pltpu.sync_copy(x_hbm.at[pl.ds(base, B_PER_W)], idx_v)
    gh = [
        pltpu.async_copy(
            table_hbm.at[idx_v.at[pl.ds(c * CHUNK, CHUNK)]], bufs[c], gsems[c])
        for c in range(NCHUNKS)
    ]
    wh = []
    for c in range(NCHUNKS):
        gh[c].wait()
        _normalize_chunk(bufs[c])
        wh.append(pltpu.async_copy(
            bufs[c], out_hbm.at[pl.ds(base + c * CHUNK, CHUNK)], wsems[c]))
    for h in wh:
        h.wait()


@jax.jit
def kernel(x, table):
    mesh = plsc.VectorSubcoreMesh(core_axis_name="c", subcore_axis_name="s")
    k = functools.partial(
        pl.kernel,
        mesh=mesh,
        out_type=jax.ShapeDtypeStruct((B, D), jnp.float32),
        scratch_types=(
            [pltpu.VMEM((B_PER_W,), jnp.int32)]
            + [pltpu.VMEM((CHUNK, D), jnp.float32)] * NCHUNKS
            + [pltpu.SemaphoreType.DMA] * (2 * NCHUNKS)
        ),
    )(_sc_body)
    return k(x.astype(jnp.int32), table)


# speedup vs baseline: 1.5227x; 1.0283x over previous
import functools

import jax
import jax.numpy as jnp
from jax import lax
from jax.experimental import pallas as pl
from jax.experimental.pallas import tpu as pltpu
from jax.experimental.pallas import tpu_sc as plsc

NUM_CATEGORIES = 100000
D = 128
B = 16384
LANES = 16
VPR = D // LANES

_info = plsc.get_sparse_core_info()
NC, NS = _info.num_cores, _info.num_subcores
NW = NC * NS
B_PER_W = B // NW
CHUNK = 128
NCHUNKS = B_PER_W // CHUNK


def _rsqrt_vec(ss):
    i = lax.bitcast_convert_type(ss, jnp.int32)
    i = jnp.full((LANES,), 0x5F3759DF, jnp.int32) - lax.shift_right_logical(i, 1)
    y = lax.bitcast_convert_type(i, jnp.float32)
    half = ss * 0.5
    for _ in range(1):
        y = y * (1.5 - half * y * y)
    return y


_GATHER_DNUMS = lax.GatherDimensionNumbers(
    offset_dims=(), collapsed_slice_dims=(0,), start_index_map=(0,)
)


def _shuffle(v, idx):
    return lax.gather(
        v, idx[:, None], _GATHER_DNUMS, (1,),
        mode=lax.GatherScatterMode.PROMISE_IN_BOUNDS,
    )


def _lane_sum(v):
    iota = lax.iota(jnp.int32, LANES)
    for k in (8, 4, 2, 1):
        v = v + _shuffle(v, iota ^ k)
    return v


def _normalize_chunk(buf):
    @plsc.parallel_loop(0, CHUNK, step=1, unroll=4)
    def row(i):
        acc = jnp.zeros((LANES,), jnp.float32)
        vs = []
        for j in range(VPR):
            v = buf[i, pl.ds(j * LANES, LANES)]
            vs.append(v)
            acc = acc + v * v
        ss = _lane_sum(acc)
        inv = _rsqrt_vec(ss)
        for j in range(VPR):
            buf[i, pl.ds(j * LANES, LANES)] = vs[j] * inv


def _sc_body(x_hbm, table_hbm, out_hbm, idx_v, *bufs_and_sems):
    bufs = bufs_and_sems[:NCHUNKS]
    gsems = bufs_and_sems[NCHUNKS:2 * NCHUNKS]
    wsems = bufs_and_sems[2 * NCHUNKS:]
    wid = lax.axis_index("s") * NC + lax.axis_index("c")
    base = wid * B_PER_W
    pltpu.sync_copy(x_hbm.at[pl.ds(base, B_PER_W)], idx_v)
    gh = [
        pltpu.async_copy(
            table_hbm.at[idx_v.at[pl.ds(c * CHUNK, CHUNK)]], bufs[c], gsems[c])
        for c in range(NCHUNKS)
    ]
    wh = []
    for c in range(NCHUNKS):
        gh[c].wait()
        _normalize_chunk(bufs[c])
        wh.append(pltpu.async_copy(
            bufs[c], out_hbm.at[pl.ds(base + c * CHUNK, CHUNK)], wsems[c]))
    for h in wh:
        h.wait()


@jax.jit
def kernel(x, table):
    mesh = plsc.VectorSubcoreMesh(core_axis_name="c", subcore_axis_name="s")
    k = functools.partial(
        pl.kernel,
        mesh=mesh,
        out_type=jax.ShapeDtypeStruct((B, D), jnp.float32),
        scratch_types=(
            [pltpu.VMEM((B_PER_W,), jnp.int32)]
            + [pltpu.VMEM((CHUNK, D), jnp.float32)] * NCHUNKS
            + [pltpu.SemaphoreType.DMA] * (2 * NCHUNKS)
        ),
    )(_sc_body)
    return k(x.astype(jnp.int32), table)
